# baseline (speedup 1.0000x reference)
import jax
import jax.numpy as jnp
from jax import lax
from jax.experimental import pallas as pl
from jax.experimental.pallas import tpu as pltpu
from jax.experimental.pallas import tpu_sc as plsc

B = 1024
K = 50
KP = 56
D = 1032
L = 16
NC = 2
NS = 16
NW = NC * NS
SPB = B // NW
NFULL = D // L
TAIL_OFF = D - L


def _sc_body(fi0, fv0, fi1, fv1, w, bias, out0, out1,
             idxv, valv, rows, biasv, acc, obuf, gsem):
    wid = lax.axis_index("s") * NC + lax.axis_index("c")
    base = wid * SPB

    pltpu.sync_copy(bias, biasv)

    def run_side(fi, fv, out):
        pltpu.sync_copy(fi.at[pl.ds(base, SPB)], idxv)
        pltpu.sync_copy(fv.at[pl.ds(base * KP, SPB * KP)], valv)

        def group_body(g, _):
            for j in range(8):
                i = g * 8 + j
                pltpu.async_copy(w.at[idxv.at[i]], rows, gsem).wait()
                for c in range(NFULL):
                    acc[pl.ds(c * L, L)] = biasv[pl.ds(c * L, L)]
                acc[pl.ds(NFULL * L, L)] = biasv[pl.ds(TAIL_OFF, L)]

                vbase = jnp.full((L,), i * KP, jnp.int32)

                def k_body(k, _):
                    v = plsc.load_gather(valv, [vbase + k])
                    for c in range(NFULL):
                        plsc.addupdate(acc.at[pl.ds(c * L, L)],
                                       v * rows[k, pl.ds(c * L, L)])
                    plsc.addupdate(acc.at[pl.ds(NFULL * L, L)],
                                   v * rows[k, pl.ds(TAIL_OFF, L)])
                    return 0

                lax.fori_loop(0, K, k_body, 0)
                for c in range(NFULL):
                    obuf[j, pl.ds(c * L, L)] = acc[pl.ds(c * L, L)]
                obuf[j, pl.ds(TAIL_OFF, L)] = acc[pl.ds(NFULL * L, L)]
            gb = pl.multiple_of(base + g * 8, 8)
            pltpu.sync_copy(obuf, out.at[pl.ds(gb, 8)])
            return 0

        lax.fori_loop(0, SPB // 8, group_body, 0)

    run_side(fi0, fv0, out0)
    run_side(fi1, fv1, out1)


@jax.jit
def _transform(fi0, fv0, fi1, fv1, w, merged_bias):
    f32 = jnp.float32
    mesh = plsc.VectorSubcoreMesh(core_axis_name="c", subcore_axis_name="s")
    out0, out1 = pl.kernel(
        _sc_body,
        out_type=(jax.ShapeDtypeStruct((B, D), f32),
                  jax.ShapeDtypeStruct((B, D), f32)),
        mesh=mesh,
        compiler_params=pltpu.CompilerParams(
            needs_layout_passes=False, use_tc_tiling_on_sc=False),
        scratch_types=[
            pltpu.VMEM((SPB, KP), jnp.int32),
            pltpu.VMEM((SPB * KP,), f32),
            pltpu.VMEM((KP, D), f32),
            pltpu.VMEM((D,), f32),
            pltpu.VMEM((NFULL * L + L,), f32),
            pltpu.VMEM((8, D), f32),
            pltpu.SemaphoreType.DMA,
        ],
    )(fi0, fv0, fi1, fv1, w, merged_bias)
    return out0, out1


def kernel(feature_indices_0, feature_values_0, feature_indices_1,
           feature_values_1, weight, bias_ft, bias_psqt):
    pad = ((0, 0), (0, KP - K))
    fi0 = jnp.pad(feature_indices_0, pad)
    fi1 = jnp.pad(feature_indices_1, pad)
    fv0 = jnp.pad(feature_values_0, pad).reshape(-1)
    fv1 = jnp.pad(feature_values_1, pad).reshape(-1)
    merged_bias = jnp.concatenate([bias_ft, bias_psqt], axis=0)
    return _transform(fi0, fv0, fi1, fv1, weight, merged_bias)


# speedup vs baseline: 1.0524x; 1.0524x over previous
import jax
import jax.numpy as jnp
from jax import lax
from jax.experimental import pallas as pl
from jax.experimental.pallas import tpu as pltpu
from jax.experimental.pallas import tpu_sc as plsc

B = 1024
K = 50
KP = 56
D = 1032
L = 16
NC = 2
NS = 16
NW = NC * NS
SPB = B // NW
NFULL = D // L
TAIL_OFF = D - L


def _sc_body(fi0, fv0, fi1, fv1, w, bias, out0, out1,
             idxv, valv, rows0, rows1, biasv, acc, obuf, sem0, sem1):
    wid = lax.axis_index("s") * NC + lax.axis_index("c")
    base = wid * SPB

    pltpu.sync_copy(bias, biasv)

    def run_side(fi, fv, out):
        pltpu.sync_copy(fi.at[pl.ds(base, SPB)], idxv)
        pltpu.sync_copy(fv.at[pl.ds(base * KP, SPB * KP)], valv)

        def gather(i, buf, sem):
            return pltpu.make_async_copy(w.at[idxv.at[i]], buf, sem)

        def compute(rbuf, i):
            for c in range(NFULL):
                acc[pl.ds(c * L, L)] = biasv[pl.ds(c * L, L)]
            acc[pl.ds(NFULL * L, L)] = biasv[pl.ds(TAIL_OFF, L)]

            vbase = jnp.full((L,), i * KP, jnp.int32)

            def k_body(k, _):
                v = plsc.load_gather(valv, [vbase + k])
                for c in range(NFULL):
                    plsc.addupdate(acc.at[pl.ds(c * L, L)],
                                   v * rbuf[k, pl.ds(c * L, L)])
                plsc.addupdate(acc.at[pl.ds(NFULL * L, L)],
                               v * rbuf[k, pl.ds(TAIL_OFF, L)])
                return 0

            lax.fori_loop(0, K, k_body, 0)
            j = lax.rem(i, 8)
            for c in range(NFULL):
                obuf[j, pl.ds(c * L, L)] = acc[pl.ds(c * L, L)]
            obuf[j, pl.ds(TAIL_OFF, L)] = acc[pl.ds(NFULL * L, L)]

            @pl.when(j == 7)
            def _():
                gb = pl.multiple_of(base + i - 7, 8)
                pltpu.sync_copy(obuf, out.at[pl.ds(gb, 8)])

        gather(0, rows0, sem0).start()

        def body(j, _):
            i = 2 * j
            gather(i + 1, rows1, sem1).start()
            gather(i, rows0, sem0).wait()
            compute(rows0, i)

            @pl.when(j < SPB // 2 - 1)
            def _():
                gather(i + 2, rows0, sem0).start()

            gather(i + 1, rows1, sem1).wait()
            compute(rows1, i + 1)
            return 0

        lax.fori_loop(0, SPB // 2, body, 0)

    run_side(fi0, fv0, out0)
    run_side(fi1, fv1, out1)


@jax.jit
def _transform(fi0, fv0, fi1, fv1, w, merged_bias):
    f32 = jnp.float32
    mesh = plsc.VectorSubcoreMesh(core_axis_name="c", subcore_axis_name="s")
    out0, out1 = pl.kernel(
        _sc_body,
        out_type=(jax.ShapeDtypeStruct((B, D), f32),
                  jax.ShapeDtypeStruct((B, D), f32)),
        mesh=mesh,
        compiler_params=pltpu.CompilerParams(
            needs_layout_passes=False, use_tc_tiling_on_sc=False),
        scratch_types=[
            pltpu.VMEM((SPB, KP), jnp.int32),
            pltpu.VMEM((SPB * KP,), f32),
            pltpu.VMEM((KP, D), f32),
            pltpu.VMEM((KP, D), f32),
            pltpu.VMEM((D,), f32),
            pltpu.VMEM((NFULL * L + L,), f32),
            pltpu.VMEM((8, D), f32),
            pltpu.SemaphoreType.DMA,
            pltpu.SemaphoreType.DMA,
        ],
    )(fi0, fv0, fi1, fv1, w, merged_bias)
    return out0, out1


def kernel(feature_indices_0, feature_values_0, feature_indices_1,
           feature_values_1, weight, bias_ft, bias_psqt):
    pad = ((0, 0), (0, KP - K))
    fi0 = jnp.pad(feature_indices_0, pad)
    fi1 = jnp.pad(feature_indices_1, pad)
    fv0 = jnp.pad(feature_values_0, pad).reshape(-1)
    fv1 = jnp.pad(feature_values_1, pad).reshape(-1)
    merged_bias = jnp.concatenate([bias_ft, bias_psqt], axis=0)
    return _transform(fi0, fv0, fi1, fv1, weight, merged_bias)


# speedup vs baseline: 1.8245x; 1.7336x over previous
import jax
import jax.numpy as jnp
from jax import lax
from jax.experimental import pallas as pl
from jax.experimental.pallas import tpu as pltpu
from jax.experimental.pallas import tpu_sc as plsc

NROWS = 100000
TBLK = 512
B = 1024
K = 50
KP = 56
D = 1032
L = 16
NC = 2
NS = 16
NW = NC * NS
SPB = B // NW
NFULL = D // L
TAIL_OFF = D - L


def _sc_body(fi0, fv0, fi1, fv1, w, bias, out0, out1,
             idxv, valv, rows0, rows1, biasv, acc, obuf, sem0, sem1):
    wid = lax.axis_index("s") * NC + lax.axis_index("c")
    base = wid * SPB

    pltpu.sync_copy(bias, biasv)

    def run_side(fi, fv, out):
        pltpu.sync_copy(fi.at[pl.ds(base, SPB)], idxv)
        pltpu.sync_copy(fv.at[pl.ds(base * KP, SPB * KP)], valv)

        def gather(i, buf, sem):
            return pltpu.make_async_copy(w.at[idxv.at[i]], buf, sem)

        def compute(rbuf, i):
            for c in range(NFULL):
                acc[pl.ds(c * L, L)] = biasv[pl.ds(c * L, L)]
            acc[pl.ds(NFULL * L, L)] = biasv[pl.ds(TAIL_OFF, L)]

            vbase = jnp.full((L,), i * KP, jnp.int32)

            def k_body(k, _):
                v = plsc.load_gather(valv, [vbase + k])
                for c in range(NFULL):
                    plsc.addupdate(acc.at[pl.ds(c * L, L)],
                                   v * rbuf[k, pl.ds(c * L, L)])
                plsc.addupdate(acc.at[pl.ds(NFULL * L, L)],
                               v * rbuf[k, pl.ds(TAIL_OFF, L)])
                return 0

            lax.fori_loop(0, K, k_body, 0)
            j = lax.rem(i, 8)
            for c in range(NFULL):
                obuf[j, pl.ds(c * L, L)] = acc[pl.ds(c * L, L)]
            obuf[j, pl.ds(TAIL_OFF, L)] = acc[pl.ds(NFULL * L, L)]

            @pl.when(j == 7)
            def _():
                gb = pl.multiple_of(base + i - 7, 8)
                pltpu.sync_copy(obuf, out.at[pl.ds(gb, 8)])

        gather(0, rows0, sem0).start()

        def body(j, _):
            i = 2 * j
            gather(i + 1, rows1, sem1).start()
            gather(i, rows0, sem0).wait()
            compute(rows0, i)

            @pl.when(j < SPB // 2 - 1)
            def _():
                gather(i + 2, rows0, sem0).start()

            gather(i + 1, rows1, sem1).wait()
            compute(rows1, i + 1)
            return 0

        lax.fori_loop(0, SPB // 2, body, 0)

    run_side(fi0, fv0, out0)
    run_side(fi1, fv1, out1)


@jax.jit
def _transform(fi0, fv0, fi1, fv1, w, merged_bias):
    f32 = jnp.float32
    mesh = plsc.VectorSubcoreMesh(core_axis_name="c", subcore_axis_name="s")
    out0, out1 = pl.kernel(
        _sc_body,
        out_type=(jax.ShapeDtypeStruct((B, D), f32),
                  jax.ShapeDtypeStruct((B, D), f32)),
        mesh=mesh,
        compiler_params=pltpu.CompilerParams(
            needs_layout_passes=False, use_tc_tiling_on_sc=False),
        scratch_types=[
            pltpu.VMEM((SPB, KP), jnp.int32),
            pltpu.VMEM((SPB * KP,), f32),
            pltpu.VMEM((KP, D), f32),
            pltpu.VMEM((KP, D), f32),
            pltpu.VMEM((D,), f32),
            pltpu.VMEM((NFULL * L + L,), f32),
            pltpu.VMEM((8, D), f32),
            pltpu.SemaphoreType.DMA,
            pltpu.SemaphoreType.DMA,
        ],
    )(fi0, fv0, fi1, fv1, w, merged_bias)
    return out0, out1


def _tp_body(in_ref, out_ref):
    out_ref[...] = in_ref[...].T


def _to_row_major(wT):
    return pl.pallas_call(
        _tp_body,
        grid=(pl.cdiv(NROWS, TBLK),),
        in_specs=[pl.BlockSpec((D, TBLK), lambda i: (0, i))],
        out_specs=pl.BlockSpec((TBLK, D), lambda i: (i, 0)),
        out_shape=jax.ShapeDtypeStruct((NROWS, D), jnp.float32),
    )(wT)


def kernel(feature_indices_0, feature_values_0, feature_indices_1,
           feature_values_1, weight, bias_ft, bias_psqt):
    pad = ((0, 0), (0, KP - K))
    fi0 = jnp.pad(feature_indices_0, pad)
    fi1 = jnp.pad(feature_indices_1, pad)
    fv0 = jnp.pad(feature_values_0, pad).reshape(-1)
    fv1 = jnp.pad(feature_values_1, pad).reshape(-1)
    merged_bias = jnp.concatenate([bias_ft, bias_psqt], axis=0)
    w_rm = _to_row_major(weight.T)
    return _transform(fi0, fv0, fi1, fv1, w_rm, merged_bias)


# speedup vs baseline: 2.3163x; 1.2695x over previous
import jax
import jax.numpy as jnp
from jax import lax
from jax.experimental import pallas as pl
from jax.experimental.pallas import tpu as pltpu
from jax.experimental.pallas import tpu_sc as plsc

NROWS = 100000
TBLK = 2048
B = 1024
K = 50
KP = 56
D = 1032
L = 16
NC = 2
NS = 16
NW = NC * NS
SPB = B // NW
NFULL = D // L
TAIL_OFF = D - L


def _sc_body(fi0, fv0, fi1, fv1, w, bias, out0, out1,
             idxv, valv, rows0, rows1, biasv, obuf, sem0, sem1):
    wid = lax.axis_index("s") * NC + lax.axis_index("c")
    base = wid * SPB

    pltpu.sync_copy(bias, biasv)

    def run_side(fi, fv, out):
        pltpu.sync_copy(fi.at[pl.ds(base, SPB)], idxv)
        pltpu.sync_copy(fv.at[pl.ds(base * KP, SPB * KP)], valv)

        def gather(i, buf, sem):
            return pltpu.make_async_copy(w.at[idxv.at[i]], buf, sem)

        WINDOWS = [c * L for c in range(NFULL)] + [TAIL_OFF]
        HALVES = (WINDOWS[:33], WINDOWS[33:])

        def compute(rbuf, i):
            j = lax.rem(i, 8)
            vbase = jnp.full((L,), i * KP, jnp.int32)
            for half in HALVES:
                def k_body(k, accs):
                    v = plsc.load_gather(valv, [vbase + k])
                    return tuple(a + v * rbuf[k, pl.ds(o, L)]
                                 for a, o in zip(accs, half))

                init = tuple(biasv[pl.ds(o, L)] for o in half)
                accs = lax.fori_loop(0, K, k_body, init)
                for a, o in zip(accs, half):
                    obuf[j, pl.ds(o, L)] = a

            @pl.when(j == 7)
            def _():
                gb = pl.multiple_of(base + i - 7, 8)
                pltpu.sync_copy(obuf, out.at[pl.ds(gb, 8)])

        gather(0, rows0, sem0).start()

        def body(j, _):
            i = 2 * j
            gather(i + 1, rows1, sem1).start()
            gather(i, rows0, sem0).wait()
            compute(rows0, i)

            @pl.when(j < SPB // 2 - 1)
            def _():
                gather(i + 2, rows0, sem0).start()

            gather(i + 1, rows1, sem1).wait()
            compute(rows1, i + 1)
            return 0

        lax.fori_loop(0, SPB // 2, body, 0)

    run_side(fi0, fv0, out0)
    run_side(fi1, fv1, out1)


@jax.jit
def _transform(fi0, fv0, fi1, fv1, w, merged_bias):
    f32 = jnp.float32
    mesh = plsc.VectorSubcoreMesh(core_axis_name="c", subcore_axis_name="s")
    out0, out1 = pl.kernel(
        _sc_body,
        out_type=(jax.ShapeDtypeStruct((B, D), f32),
                  jax.ShapeDtypeStruct((B, D), f32)),
        mesh=mesh,
        compiler_params=pltpu.CompilerParams(
            needs_layout_passes=False, use_tc_tiling_on_sc=False),
        scratch_types=[
            pltpu.VMEM((SPB, KP), jnp.int32),
            pltpu.VMEM((SPB * KP,), f32),
            pltpu.VMEM((KP, D), f32),
            pltpu.VMEM((KP, D), f32),
            pltpu.VMEM((D,), f32),
            pltpu.VMEM((8, D), f32),
            pltpu.SemaphoreType.DMA,
            pltpu.SemaphoreType.DMA,
        ],
    )(fi0, fv0, fi1, fv1, w, merged_bias)
    return out0, out1


def _tp_body(in_ref, out_ref):
    out_ref[...] = in_ref[...].T


def _to_row_major(wT):
    return pl.pallas_call(
        _tp_body,
        grid=(pl.cdiv(NROWS, TBLK),),
        in_specs=[pl.BlockSpec((D, TBLK), lambda i: (0, i))],
        out_specs=pl.BlockSpec((TBLK, D), lambda i: (i, 0)),
        out_shape=jax.ShapeDtypeStruct((NROWS, D), jnp.float32),
    )(wT)


def kernel(feature_indices_0, feature_values_0, feature_indices_1,
           feature_values_1, weight, bias_ft, bias_psqt):
    pad = ((0, 0), (0, KP - K))
    fi0 = jnp.pad(feature_indices_0, pad)
    fi1 = jnp.pad(feature_indices_1, pad)
    fv0 = jnp.pad(feature_values_0, pad).reshape(-1)
    fv1 = jnp.pad(feature_values_1, pad).reshape(-1)
    merged_bias = jnp.concatenate([bias_ft, bias_psqt], axis=0)
    w_rm = _to_row_major(weight.T)
    return _transform(fi0, fv0, fi1, fv1, w_rm, merged_bias)
